# baseline (speedup 1.0000x reference)
import functools

import jax
import jax.numpy as jnp
from jax import lax
from jax.experimental import pallas as pl
from jax.experimental.pallas import tpu as pltpu
from jax.experimental.pallas import tpu_sc as plsc

NC = 2
NS = 16
NW = NC * NS
CHUNK = 128


def _edge_mlp_body(x_ref, eW1, eb1, eW2, eb2, y_ref):
    x = x_ref[...]
    f32 = jnp.float32
    h = jnp.maximum(jnp.dot(x, eW1[...], preferred_element_type=f32) + eb1[...], 0.0)
    y_ref[...] = jnp.dot(h, eW2[...], preferred_element_type=f32) + eb2[...]


def _grid_mlp_body(x_ref, gW1, gb1, gW2, gb2, gout_ref):
    x = x_ref[...]
    f32 = jnp.float32
    g = jnp.maximum(jnp.dot(x, gW1[...], preferred_element_type=f32) + gb1[...], 0.0)
    gout_ref[...] = x + jnp.dot(g, gW2[...], preferred_element_type=f32) + gb2[...]


def _mesh_body(a_ref, mW1, mb1, mW2, mb2, out_ref):
    x = a_ref[0] + a_ref[1]
    f32 = jnp.float32
    h = jnp.maximum(jnp.dot(x, mW1[...], preferred_element_type=f32) + mb1[...], 0.0)
    out_ref[...] = jnp.dot(h, mW2[...], preferred_element_type=f32) + mb2[...]


def _mlp_specs(br, d):
    x_spec = pl.BlockSpec((br, d), lambda i: (i, 0))
    w_spec = pl.BlockSpec((d, d), lambda i: (0, 0))
    b_spec = pl.BlockSpec((1, d), lambda i: (0, 0))
    return [x_spec, w_spec, b_spec, w_spec, b_spec]


def _row_mlp(body, x, W1, b1, W2, b2, br, x_spec=None):
    n, d = x.shape[-2:]
    specs = _mlp_specs(br, d)
    if x_spec is not None:
        specs[0] = x_spec
    return pl.pallas_call(
        body,
        grid=(n // br,),
        in_specs=specs,
        out_specs=pl.BlockSpec((br, d), lambda i: (i, 0)),
        out_shape=jax.ShapeDtypeStruct((n, d), jnp.float32),
    )(x, W1, b1.reshape(1, d), W2, b2.reshape(1, d))


def _sc_scatter_body(y_hbm, edge_hbm, zeros_hbm, out_hbm,
                     ed_c, rows_v, agg_sh, sem_g0, sem_g1, sem_i,
                     nchunk, nleft, n_mesh):
    c = lax.axis_index("c")
    s = lax.axis_index("s")
    wid = s * NC + c

    def edges_at(g):
        return edge_hbm.at[:, pl.ds(g * CHUNK, CHUNK)]

    base = (n_mesh // (NS * 8)) * 8
    last = n_mesh - base * (NS - 1)

    g0 = wid * nchunk
    pltpu.async_copy(edges_at(g0), ed_c.at[0], sem_g1)
    pltpu.async_copy(edges_at(g0 + 1), ed_c.at[1], sem_i)

    @pl.when(s < NS - 1)
    def _():
        pltpu.sync_copy(zeros_hbm.at[pl.ds(0, base)],
                        agg_sh.at[pl.ds(s * base, base)])

    @pl.when(s == NS - 1)
    def _():
        pltpu.sync_copy(zeros_hbm.at[pl.ds(0, last)],
                        agg_sh.at[pl.ds((NS - 1) * base, last)])

    plsc.subcore_barrier()

    sems = (sem_g0, sem_g1)
    pltpu.make_async_copy(edges_at(g0), ed_c.at[0], sem_g1).wait()
    pltpu.async_copy(y_hbm.at[ed_c.at[0, 0]], rows_v.at[0], sem_g0)

    def body6(jj, carry):
        for u in range(6):
            j = jj * 6 + u
            b = u % 2
            nb = 1 - b

            @pl.when(j + 1 < nchunk)
            def _():
                pltpu.make_async_copy(edges_at(g0 + j + 1),
                                      ed_c.at[(u + 1) % 3], sem_i).wait()
                pltpu.async_copy(y_hbm.at[ed_c.at[(u + 1) % 3, 0]],
                                 rows_v.at[nb], sems[nb])

            pltpu.make_async_copy(y_hbm.at[ed_c.at[u % 3, 0]], rows_v.at[b],
                                  sems[b]).wait()

            @pl.when(j + 2 < nchunk)
            def _():
                pltpu.async_copy(edges_at(g0 + j + 2), ed_c.at[(u + 2) % 3],
                                 sem_i)

            pltpu.sync_copy(rows_v.at[b], agg_sh.at[ed_c.at[u % 3, 1]],
                            add=True)
        return carry

    lax.fori_loop(0, nchunk // 6, body6, 0)

    @pl.when(wid < nleft)
    def _():
        gl = NW * nchunk + wid
        pltpu.sync_copy(edges_at(gl), ed_c.at[0])
        pltpu.async_copy(y_hbm.at[ed_c.at[0, 0]], rows_v.at[0], sem_g0).wait()
        pltpu.sync_copy(rows_v.at[0], agg_sh.at[ed_c.at[0, 1]], add=True)

    plsc.subcore_barrier()

    @pl.when(s < NS - 1)
    def _():
        pltpu.sync_copy(agg_sh.at[pl.ds(s * base, base)],
                        out_hbm.at[c, pl.ds(s * base, base)])

    @pl.when(s == NS - 1)
    def _():
        pltpu.sync_copy(agg_sh.at[pl.ds((NS - 1) * base, last)],
                        out_hbm.at[c, pl.ds((NS - 1) * base, last)])


def kernel(grid_x, mesh_x, edge_index,
           eW1, eb1, eW2, eb2,
           mW1, mb1, mW2, mb2,
           gW1, gb1, gW2, gb2):
    n_grid, d = grid_x.shape
    n_mesh = mesh_x.shape[0]
    e = edge_index.shape[1]
    total_chunks = e // CHUNK
    nchunk = total_chunks // NW
    nleft = total_chunks - nchunk * NW

    zrows = (n_mesh // (NS * 8)) * 8 + 16
    zeros = jnp.zeros((zrows, d), dtype=jnp.float32)

    br = 1000
    y = _row_mlp(_edge_mlp_body, grid_x, eW1, eb1, eW2, eb2, br)

    mesh_sc = plsc.VectorSubcoreMesh(
        core_axis_name="c", subcore_axis_name="s",
        num_cores=NC, num_subcores=NS)
    agg2 = pl.kernel(
        functools.partial(_sc_scatter_body, nchunk=nchunk, nleft=nleft,
                          n_mesh=n_mesh),
        out_type=jax.ShapeDtypeStruct((NC, n_mesh, d), jnp.float32),
        mesh=mesh_sc,
        scratch_types=[
            pltpu.VMEM((3, 2, CHUNK), jnp.int32),
            pltpu.VMEM((2, CHUNK, d), jnp.float32),
            pltpu.VMEM_SHARED((n_mesh, d), jnp.float32),
            pltpu.SemaphoreType.DMA,
            pltpu.SemaphoreType.DMA,
            pltpu.SemaphoreType.DMA,
        ],
    )(y, edge_index, zeros)

    grid_out = _row_mlp(_grid_mlp_body, grid_x, gW1, gb1, gW2, gb2, br)

    mesh_out = _row_mlp(
        _mesh_body, agg2, mW1, mb1, mW2, mb2, br,
        x_spec=pl.BlockSpec((2, br, d), lambda i: (0, i, 0)))

    return grid_out, mesh_out


# speedup vs baseline: 1.0369x; 1.0369x over previous
import functools

import jax
import jax.numpy as jnp
from jax import lax
from jax.experimental import pallas as pl
from jax.experimental.pallas import tpu as pltpu
from jax.experimental.pallas import tpu_sc as plsc

NC = 2
NS = 16
NW = NC * NS
CHUNK = 128


def _edge_mlp_body(x_ref, eW1, eb1, eW2, eb2, y_ref):
    x = x_ref[...]
    f32 = jnp.float32
    h = jnp.maximum(jnp.dot(x, eW1[...], preferred_element_type=f32) + eb1[...], 0.0)
    y_ref[...] = jnp.dot(h, eW2[...], preferred_element_type=f32) + eb2[...]


def _grid_mlp_body(x_ref, gW1, gb1, gW2, gb2, gout_ref):
    x = x_ref[...]
    f32 = jnp.float32
    g = jnp.maximum(jnp.dot(x, gW1[...], preferred_element_type=f32) + gb1[...], 0.0)
    gout_ref[...] = x + jnp.dot(g, gW2[...], preferred_element_type=f32) + gb2[...]


def _mesh_body(a_ref, mW1, mb1, mW2, mb2, out_ref):
    x = a_ref[0] + a_ref[1]
    f32 = jnp.float32
    h = jnp.maximum(jnp.dot(x, mW1[...], preferred_element_type=f32) + mb1[...], 0.0)
    out_ref[...] = jnp.dot(h, mW2[...], preferred_element_type=f32) + mb2[...]


def _mlp_specs(br, d):
    x_spec = pl.BlockSpec((br, d), lambda i: (i, 0))
    w_spec = pl.BlockSpec((d, d), lambda i: (0, 0))
    b_spec = pl.BlockSpec((1, d), lambda i: (0, 0))
    return [x_spec, w_spec, b_spec, w_spec, b_spec]


def _row_mlp(body, x, W1, b1, W2, b2, br, x_spec=None):
    n, d = x.shape[-2:]
    specs = _mlp_specs(br, d)
    if x_spec is not None:
        specs[0] = x_spec
    return pl.pallas_call(
        body,
        grid=(n // br,),
        in_specs=specs,
        out_specs=pl.BlockSpec((br, d), lambda i: (i, 0)),
        out_shape=jax.ShapeDtypeStruct((n, d), jnp.float32),
    )(x, W1, b1.reshape(1, d), W2, b2.reshape(1, d))


def _sc_scatter_body(y_hbm, edge_hbm, zeros_hbm, out_hbm,
                     ed_c, rows_v, agg_sh, sem_g0, sem_g1, sem_i,
                     nchunk, nleft, n_mesh):
    c = lax.axis_index("c")
    s = lax.axis_index("s")
    wid = s * NC + c

    def edges_at(g):
        return edge_hbm.at[:, pl.ds(g * CHUNK, CHUNK)]

    base = (n_mesh // (NS * 8)) * 8
    last = n_mesh - base * (NS - 1)

    g0 = wid * nchunk
    pltpu.async_copy(edges_at(g0), ed_c.at[0], sem_g1)
    pltpu.async_copy(edges_at(g0 + 1), ed_c.at[1], sem_i)

    @pl.when(s < NS - 1)
    def _():
        pltpu.sync_copy(zeros_hbm.at[pl.ds(0, base)],
                        agg_sh.at[pl.ds(s * base, base)])

    @pl.when(s == NS - 1)
    def _():
        pltpu.sync_copy(zeros_hbm.at[pl.ds(0, last)],
                        agg_sh.at[pl.ds((NS - 1) * base, last)])

    plsc.subcore_barrier()

    sems = (sem_g0, sem_g1)
    pltpu.make_async_copy(edges_at(g0), ed_c.at[0], sem_g1).wait()
    pltpu.async_copy(y_hbm.at[ed_c.at[0, 0]], rows_v.at[0], sem_g0)

    def body6(jj, carry):
        for u in range(6):
            j = jj * 6 + u
            b = u % 2
            nb = 1 - b

            @pl.when(j + 1 < nchunk)
            def _():
                pltpu.make_async_copy(edges_at(g0 + j + 1),
                                      ed_c.at[(u + 1) % 3], sem_i).wait()
                pltpu.async_copy(y_hbm.at[ed_c.at[(u + 1) % 3, 0]],
                                 rows_v.at[nb], sems[nb])

            pltpu.make_async_copy(y_hbm.at[ed_c.at[u % 3, 0]], rows_v.at[b],
                                  sems[b]).wait()

            @pl.when(j + 2 < nchunk)
            def _():
                pltpu.async_copy(edges_at(g0 + j + 2), ed_c.at[(u + 2) % 3],
                                 sem_i)

            pltpu.sync_copy(rows_v.at[b], agg_sh.at[ed_c.at[u % 3, 1]],
                            add=True)
        return carry

    lax.fori_loop(0, nchunk // 6, body6, 0)

    @pl.when(wid < nleft)
    def _():
        gl = NW * nchunk + wid
        pltpu.sync_copy(edges_at(gl), ed_c.at[0])
        pltpu.async_copy(y_hbm.at[ed_c.at[0, 0]], rows_v.at[0], sem_g0).wait()
        pltpu.sync_copy(rows_v.at[0], agg_sh.at[ed_c.at[0, 1]], add=True)

    plsc.subcore_barrier()

    @pl.when(s < NS - 1)
    def _():
        pltpu.sync_copy(agg_sh.at[pl.ds(s * base, base)],
                        out_hbm.at[c, pl.ds(s * base, base)])

    @pl.when(s == NS - 1)
    def _():
        pltpu.sync_copy(agg_sh.at[pl.ds((NS - 1) * base, last)],
                        out_hbm.at[c, pl.ds((NS - 1) * base, last)])


def kernel(grid_x, mesh_x, edge_index,
           eW1, eb1, eW2, eb2,
           mW1, mb1, mW2, mb2,
           gW1, gb1, gW2, gb2):
    n_grid, d = grid_x.shape
    n_mesh = mesh_x.shape[0]
    e = edge_index.shape[1]
    total_chunks = e // CHUNK
    nchunk = total_chunks // NW
    nleft = total_chunks - nchunk * NW

    zrows = (n_mesh // (NS * 8)) * 8 + 16
    zeros = jnp.zeros((zrows, d), dtype=jnp.float32)

    br = 2000
    y = _row_mlp(_edge_mlp_body, grid_x, eW1, eb1, eW2, eb2, br)

    mesh_sc = plsc.VectorSubcoreMesh(
        core_axis_name="c", subcore_axis_name="s",
        num_cores=NC, num_subcores=NS)
    agg2 = pl.kernel(
        functools.partial(_sc_scatter_body, nchunk=nchunk, nleft=nleft,
                          n_mesh=n_mesh),
        out_type=jax.ShapeDtypeStruct((NC, n_mesh, d), jnp.float32),
        mesh=mesh_sc,
        scratch_types=[
            pltpu.VMEM((3, 2, CHUNK), jnp.int32),
            pltpu.VMEM((2, CHUNK, d), jnp.float32),
            pltpu.VMEM_SHARED((n_mesh, d), jnp.float32),
            pltpu.SemaphoreType.DMA,
            pltpu.SemaphoreType.DMA,
            pltpu.SemaphoreType.DMA,
        ],
    )(y, edge_index, zeros)

    grid_out = _row_mlp(_grid_mlp_body, grid_x, gW1, gb1, gW2, gb2, br)

    mesh_out = _row_mlp(
        _mesh_body, agg2, mW1, mb1, mW2, mb2, br,
        x_spec=pl.BlockSpec((2, br, d), lambda i: (0, i, 0)))

    return grid_out, mesh_out


# speedup vs baseline: 1.0541x; 1.0167x over previous
import functools

import jax
import jax.numpy as jnp
from jax import lax
from jax.experimental import pallas as pl
from jax.experimental.pallas import tpu as pltpu
from jax.experimental.pallas import tpu_sc as plsc

NC = 2
NS = 16
NW = NC * NS
CHUNK = 128


def _edge_mlp_body(x_ref, eW1, eb1, eW2, eb2, y_ref):
    x = x_ref[...]
    f32 = jnp.float32
    h = jnp.maximum(jnp.dot(x, eW1[...], preferred_element_type=f32) + eb1[...], 0.0)
    y_ref[...] = jnp.dot(h, eW2[...], preferred_element_type=f32) + eb2[...]


def _grid_mlp_body(x_ref, gW1, gb1, gW2, gb2, gout_ref):
    x = x_ref[...]
    f32 = jnp.float32
    g = jnp.maximum(jnp.dot(x, gW1[...], preferred_element_type=f32) + gb1[...], 0.0)
    gout_ref[...] = x + jnp.dot(g, gW2[...], preferred_element_type=f32) + gb2[...]


def _mesh_body(a_ref, mW1, mb1, mW2, mb2, out_ref):
    x = a_ref[0] + a_ref[1]
    f32 = jnp.float32
    h = jnp.maximum(jnp.dot(x, mW1[...], preferred_element_type=f32) + mb1[...], 0.0)
    out_ref[...] = jnp.dot(h, mW2[...], preferred_element_type=f32) + mb2[...]


def _mlp_specs(br, d):
    x_spec = pl.BlockSpec((br, d), lambda i: (i, 0))
    w_spec = pl.BlockSpec((d, d), lambda i: (0, 0))
    b_spec = pl.BlockSpec((1, d), lambda i: (0, 0))
    return [x_spec, w_spec, b_spec, w_spec, b_spec]


def _row_mlp(body, x, W1, b1, W2, b2, br, x_spec=None):
    n, d = x.shape[-2:]
    specs = _mlp_specs(br, d)
    if x_spec is not None:
        specs[0] = x_spec
    return pl.pallas_call(
        body,
        grid=(n // br,),
        in_specs=specs,
        out_specs=pl.BlockSpec((br, d), lambda i: (i, 0)),
        out_shape=jax.ShapeDtypeStruct((n, d), jnp.float32),
    )(x, W1, b1.reshape(1, d), W2, b2.reshape(1, d))


def _sc_scatter_body(y_hbm, edge_hbm, zeros_hbm, out_hbm,
                     ed_c, rows_v, agg_sh, sem_g0, sem_g1, sem_i,
                     nchunk, nleft, n_mesh):
    c = lax.axis_index("c")
    s = lax.axis_index("s")
    wid = s * NC + c

    def edges_at(g):
        return edge_hbm.at[:, pl.ds(g * CHUNK, CHUNK)]

    base = (n_mesh // (NS * 8)) * 8
    last = n_mesh - base * (NS - 1)

    g0 = wid * nchunk
    pltpu.async_copy(edges_at(g0), ed_c.at[0], sem_g1)
    pltpu.async_copy(edges_at(g0 + 1), ed_c.at[1], sem_i)

    @pl.when(s < NS - 1)
    def _():
        pltpu.sync_copy(zeros_hbm.at[pl.ds(0, base)],
                        agg_sh.at[pl.ds(s * base, base)])

    @pl.when(s == NS - 1)
    def _():
        pltpu.sync_copy(zeros_hbm.at[pl.ds(0, last)],
                        agg_sh.at[pl.ds((NS - 1) * base, last)])

    plsc.subcore_barrier()

    sems = (sem_g0, sem_g1)
    pltpu.make_async_copy(edges_at(g0), ed_c.at[0], sem_g1).wait()
    pltpu.async_copy(y_hbm.at[ed_c.at[0, 0]], rows_v.at[0], sem_g0)

    def body6(jj, carry):
        for u in range(6):
            j = jj * 6 + u
            b = u % 2
            nb = 1 - b

            @pl.when(j + 1 < nchunk)
            def _():
                pltpu.make_async_copy(edges_at(g0 + j + 1),
                                      ed_c.at[(u + 1) % 3], sem_i).wait()
                pltpu.async_copy(y_hbm.at[ed_c.at[(u + 1) % 3, 0]],
                                 rows_v.at[nb], sems[nb])

            pltpu.make_async_copy(y_hbm.at[ed_c.at[u % 3, 0]], rows_v.at[b],
                                  sems[b]).wait()

            @pl.when(j + 2 < nchunk)
            def _():
                pltpu.async_copy(edges_at(g0 + j + 2), ed_c.at[(u + 2) % 3],
                                 sem_i)

            pltpu.sync_copy(rows_v.at[b], agg_sh.at[ed_c.at[u % 3, 1]],
                            add=True)
        return carry

    lax.fori_loop(0, nchunk // 6, body6, 0)

    @pl.when(wid < nleft)
    def _():
        gl = NW * nchunk + wid
        pltpu.sync_copy(edges_at(gl), ed_c.at[0])
        pltpu.async_copy(y_hbm.at[ed_c.at[0, 0]], rows_v.at[0], sem_g0).wait()
        pltpu.sync_copy(rows_v.at[0], agg_sh.at[ed_c.at[0, 1]], add=True)

    plsc.subcore_barrier()

    @pl.when(s < NS - 1)
    def _():
        pltpu.sync_copy(agg_sh.at[pl.ds(s * base, base)],
                        out_hbm.at[c, pl.ds(s * base, base)])

    @pl.when(s == NS - 1)
    def _():
        pltpu.sync_copy(agg_sh.at[pl.ds((NS - 1) * base, last)],
                        out_hbm.at[c, pl.ds((NS - 1) * base, last)])


def kernel(grid_x, mesh_x, edge_index,
           eW1, eb1, eW2, eb2,
           mW1, mb1, mW2, mb2,
           gW1, gb1, gW2, gb2):
    n_grid, d = grid_x.shape
    n_mesh = mesh_x.shape[0]
    e = edge_index.shape[1]
    total_chunks = e // CHUNK
    nchunk = total_chunks // NW
    nleft = total_chunks - nchunk * NW

    zrows = (n_mesh // (NS * 8)) * 8 + 16
    zeros = jnp.zeros((zrows, d), dtype=jnp.float32)

    br = 5000
    y = _row_mlp(_edge_mlp_body, grid_x, eW1, eb1, eW2, eb2, br)

    mesh_sc = plsc.VectorSubcoreMesh(
        core_axis_name="c", subcore_axis_name="s",
        num_cores=NC, num_subcores=NS)
    agg2 = pl.kernel(
        functools.partial(_sc_scatter_body, nchunk=nchunk, nleft=nleft,
                          n_mesh=n_mesh),
        out_type=jax.ShapeDtypeStruct((NC, n_mesh, d), jnp.float32),
        mesh=mesh_sc,
        scratch_types=[
            pltpu.VMEM((3, 2, CHUNK), jnp.int32),
            pltpu.VMEM((2, CHUNK, d), jnp.float32),
            pltpu.VMEM_SHARED((n_mesh, d), jnp.float32),
            pltpu.SemaphoreType.DMA,
            pltpu.SemaphoreType.DMA,
            pltpu.SemaphoreType.DMA,
        ],
    )(y, edge_index, zeros)

    grid_out = _row_mlp(_grid_mlp_body, grid_x, gW1, gb1, gW2, gb2, br)

    mesh_out = _row_mlp(
        _mesh_body, agg2, mW1, mb1, mW2, mb2, br,
        x_spec=pl.BlockSpec((2, br, d), lambda i: (0, i, 0)))

    return grid_out, mesh_out
